# Initial kernel scaffold; baseline (speedup 1.0000x reference)
import jax
import jax.numpy as jnp
from jax.experimental import pallas as pl


def kernel(x, edge_index, W1, b1, g1, be1, W2, b2, g2, be2, W3, b3):
    raise NotImplementedError("write your pallas kernel here")


# speedup vs baseline: 6.8106x; 6.8106x over previous
import functools

import jax
import jax.numpy as jnp
import numpy as np
from jax import lax
from jax.experimental import pallas as pl
from jax.experimental.pallas import tpu as pltpu
from jax.experimental.pallas import tpu_sc as plsc

N = 10000
E = 160000
D_IN = 256
D_H = 256
D_OUT = 128
BN_EPS = 1e-5

NC = 2
NS = 16
STRIPE = 1000
NSTRIPES = N // STRIPE
EDGES_PER_TILE = E // NS
EDGES_PER_WORKER = E // (NC * NS)
CHUNK = 80
CHUNK3 = 40
DEG_CHUNK = 40


def _sc_mesh():
  return plsc.VectorSubcoreMesh(
      core_axis_name="c", subcore_axis_name="s", num_cores=NC, num_subcores=NS
  )


@functools.lru_cache(maxsize=None)
def _make_deg_kernel():
  n_chunks = EDGES_PER_WORKER // DEG_CHUNK
  edges_per_core = E // NC

  @functools.partial(
      pl.kernel,
      out_type=jax.ShapeDtypeStruct((NC, N, 128), jnp.float32),
      mesh=_sc_mesh(),
      scratch_types=[
          pltpu.VMEM((DEG_CHUNK,), jnp.int32),
          pltpu.VMEM((DEG_CHUNK, 128), jnp.float32),
          pltpu.VMEM_SHARED((N, 128), jnp.float32),
      ],
  )
  def deg_kernel(dst_hbm, zeros_hbm, ones_hbm, out_hbm, idx_d, ones_v, acc):
    c = lax.axis_index("c")
    s = lax.axis_index("s")
    pltpu.sync_copy(ones_hbm, ones_v)

    @pl.when(s < NSTRIPES)
    def _init():
      pltpu.sync_copy(
          zeros_hbm.at[pl.ds(s * STRIPE, STRIPE)],
          acc.at[pl.ds(s * STRIPE, STRIPE)],
      )

    plsc.subcore_barrier()
    base = c * edges_per_core + s * EDGES_PER_WORKER

    def body(j, carry):
      pltpu.sync_copy(dst_hbm.at[pl.ds(base + j * DEG_CHUNK, DEG_CHUNK)], idx_d)
      pltpu.sync_copy(ones_v, acc.at[idx_d], add=True)
      return carry

    lax.fori_loop(0, n_chunks, body, 0)
    plsc.subcore_barrier()

    @pl.when(s < NSTRIPES)
    def _out():
      pltpu.sync_copy(
          acc.at[pl.ds(s * STRIPE, STRIPE)],
          out_hbm.at[c, pl.ds(s * STRIPE, STRIPE)],
      )

  return deg_kernel


@functools.lru_cache(maxsize=None)
def _make_agg_kernel(half):
  n_chunks = EDGES_PER_TILE // CHUNK

  @functools.partial(
      pl.kernel,
      out_type=jax.ShapeDtypeStruct((NC, N, half), jnp.float32),
      mesh=_sc_mesh(),
      scratch_types=[
          pltpu.VMEM((CHUNK,), jnp.int32),
          pltpu.VMEM((CHUNK,), jnp.int32),
          pltpu.VMEM((CHUNK, half), jnp.float32),
          pltpu.VMEM_SHARED((N, half), jnp.float32),
          pltpu.SemaphoreType.DMA,
      ],
  )
  def agg_kernel(srcc_hbm, dst_hbm, hs_hbm, out_hbm, idx_s, idx_d, rows, acc, sem):
    c = lax.axis_index("c")
    s = lax.axis_index("s")
    @pl.when(s < NSTRIPES)
    def _init():
      pltpu.sync_copy(
          hs_hbm.at[pl.ds(c * N + s * STRIPE, STRIPE)],
          acc.at[pl.ds(s * STRIPE, STRIPE)],
      )

    plsc.subcore_barrier()
    ebase = s * EDGES_PER_TILE

    def body(j, carry):
      off = ebase + j * CHUNK
      pltpu.sync_copy(srcc_hbm.at[pl.ds(c * E + off, CHUNK)], idx_s)
      pltpu.sync_copy(dst_hbm.at[pl.ds(off, CHUNK)], idx_d)
      pltpu.async_copy(hs_hbm.at[idx_s], rows, sem).wait()
      pltpu.sync_copy(rows, acc.at[idx_d], add=True)
      return carry

    lax.fori_loop(0, n_chunks, body, 0)
    plsc.subcore_barrier()

    @pl.when(s < NSTRIPES)
    def _out():
      pltpu.sync_copy(
          acc.at[pl.ds(s * STRIPE, STRIPE)],
          out_hbm.at[c, pl.ds(s * STRIPE, STRIPE)],
      )

  return agg_kernel


@functools.lru_cache(maxsize=None)
def _make_agg3_kernel():
  edges_per_core = E // NC
  edges_per_tile3 = edges_per_core // NS
  n_chunks = edges_per_tile3 // CHUNK3

  @functools.partial(
      pl.kernel,
      out_type=jax.ShapeDtypeStruct((NC, N, D_OUT), jnp.float32),
      mesh=_sc_mesh(),
      scratch_types=[
          pltpu.VMEM((CHUNK3,), jnp.int32),
          pltpu.VMEM((CHUNK3,), jnp.int32),
          pltpu.VMEM((CHUNK3, D_OUT), jnp.float32),
          pltpu.VMEM_SHARED((N, D_OUT), jnp.float32),
          pltpu.SemaphoreType.DMA,
      ],
  )
  def agg3_kernel(src_hbm, dst_hbm, hs_hbm, out_hbm, idx_s, idx_d, rows, acc, sem):
    c = lax.axis_index("c")
    s = lax.axis_index("s")

    @pl.when(s < NSTRIPES)
    def _init():
      pltpu.sync_copy(
          hs_hbm.at[pl.ds(s * STRIPE, STRIPE)],
          acc.at[pl.ds(s * STRIPE, STRIPE)],
      )

    plsc.subcore_barrier()
    ebase = c * edges_per_core + s * edges_per_tile3

    def body(j, carry):
      off = ebase + j * CHUNK3
      pltpu.sync_copy(src_hbm.at[pl.ds(off, CHUNK3)], idx_s)
      pltpu.sync_copy(dst_hbm.at[pl.ds(off, CHUNK3)], idx_d)
      pltpu.async_copy(hs_hbm.at[idx_s], rows, sem).wait()
      pltpu.sync_copy(rows, acc.at[idx_d], add=True)
      return carry

    lax.fori_loop(0, n_chunks, body, 0)
    plsc.subcore_barrier()

    @pl.when(s < NSTRIPES)
    def _out():
      pltpu.sync_copy(
          acc.at[pl.ds(s * STRIPE, STRIPE)],
          out_hbm.at[c, pl.ds(s * STRIPE, STRIPE)],
      )

  return agg3_kernel


_BN_ROWS = 1000


def _k1_body(x_ref, w_ref, parts_ref, hs_ref, dinv_ref):
  deg = parts_ref[0, :, 0:1] + parts_ref[1, :, 0:1] + 1.0
  dv = lax.rsqrt(jnp.maximum(deg, 1.0))
  h = lax.dot_general(
      x_ref[...], w_ref[...], (((1,), (0,)), ((), ())),
      preferred_element_type=jnp.float32,
  )
  hs = h * dv
  hs_ref[0] = hs[:, :128]
  hs_ref[1] = hs[:, 128:]
  dinv_ref[...] = dv


def _k1(x, w1, parts):
  g = N // _BN_ROWS
  return pl.pallas_call(
      _k1_body,
      grid=(g,),
      in_specs=[
          pl.BlockSpec((_BN_ROWS, D_IN), lambda i: (i, 0)),
          pl.BlockSpec((D_IN, D_H), lambda i: (0, 0)),
          pl.BlockSpec((NC, _BN_ROWS, 128), lambda i: (0, i, 0)),
      ],
      out_specs=[
          pl.BlockSpec((NC, _BN_ROWS, D_H // 2), lambda i: (0, i, 0)),
          pl.BlockSpec((_BN_ROWS, 1), lambda i: (i, 0)),
      ],
      out_shape=[
          jax.ShapeDtypeStruct((NC, N, D_H // 2), jnp.float32),
          jax.ShapeDtypeStruct((N, 1), jnp.float32),
      ],
  )(x, w1, parts)


def _make_mid_body(d_out, split):
  half = d_out // 2
  bn_c = float(1.0 / np.sqrt(np.float32(1.0 + BN_EPS), dtype=np.float32))

  def body(agg_ref, dinv_ref, b_ref, g_ref, be_ref, w_ref, hs_ref):
    dv = dinv_ref[...]
    a = jnp.concatenate([agg_ref[0], agg_ref[1]], axis=1)
    prev = a * dv + b_ref[...]
    t = prev * (g_ref[...] * bn_c) + be_ref[...]
    h = jnp.maximum(t, 0.0)
    hs = lax.dot_general(
        h, w_ref[...], (((1,), (0,)), ((), ())),
        preferred_element_type=jnp.float32,
    ) * dv
    if split:
      hs_ref[0] = hs[:, :half]
      hs_ref[1] = hs[:, half:]
    else:
      hs_ref[...] = hs

  return body


def _mid_layer(agg, dinv, b, gm, be, w, d_out, split=True):
  g = N // _BN_ROWS
  half = d_out // 2
  if split:
    out_specs = pl.BlockSpec((NC, _BN_ROWS, half), lambda i: (0, i, 0))
    out_shape = jax.ShapeDtypeStruct((NC, N, half), jnp.float32)
  else:
    out_specs = pl.BlockSpec((_BN_ROWS, d_out), lambda i: (i, 0))
    out_shape = jax.ShapeDtypeStruct((N, d_out), jnp.float32)
  return pl.pallas_call(
      _make_mid_body(d_out, split),
      grid=(g,),
      in_specs=[
          pl.BlockSpec((NC, _BN_ROWS, D_H // 2), lambda i: (0, i, 0)),
          pl.BlockSpec((_BN_ROWS, 1), lambda i: (i, 0)),
          pl.BlockSpec((1, D_H), lambda i: (0, 0)),
          pl.BlockSpec((1, D_H), lambda i: (0, 0)),
          pl.BlockSpec((1, D_H), lambda i: (0, 0)),
          pl.BlockSpec((D_H, d_out), lambda i: (0, 0)),
      ],
      out_specs=out_specs,
      out_shape=out_shape,
  )(agg, dinv, b.reshape(1, -1), gm.reshape(1, -1), be.reshape(1, -1), w)


def _k4_body(agg_ref, hs3_ref, dinv_ref, b_ref, out_ref):
  a = agg_ref[0] + agg_ref[1] - hs3_ref[...]
  out_ref[...] = a * dinv_ref[...] + b_ref[...]


def _k4(agg, hs3, dinv, b3):
  g = N // _BN_ROWS
  return pl.pallas_call(
      _k4_body,
      grid=(g,),
      in_specs=[
          pl.BlockSpec((NC, _BN_ROWS, D_OUT), lambda i: (0, i, 0)),
          pl.BlockSpec((_BN_ROWS, D_OUT), lambda i: (i, 0)),
          pl.BlockSpec((_BN_ROWS, 1), lambda i: (i, 0)),
          pl.BlockSpec((1, D_OUT), lambda i: (0, 0)),
      ],
      out_specs=pl.BlockSpec((_BN_ROWS, D_OUT), lambda i: (i, 0)),
      out_shape=jax.ShapeDtypeStruct((N, D_OUT), jnp.float32),
  )(agg, hs3, dinv, b3.reshape(1, -1))


def kernel(x, edge_index, W1, b1, g1, be1, W2, b2, g2, be2, W3, b3):
  src = edge_index[0]
  dst = edge_index[1]
  srcc = jnp.concatenate([src, src + N])
  zeros128 = jnp.zeros((N, 128), jnp.float32)
  ones128 = jnp.ones((DEG_CHUNK, 128), jnp.float32)

  parts = _make_deg_kernel()(dst, zeros128, ones128)
  hs1, dinv = _k1(x, W1, parts)
  agg1 = _make_agg_kernel(D_H // 2)(srcc, dst, hs1.reshape(NC * N, D_H // 2))
  hs2 = _mid_layer(agg1, dinv, b1, g1, be1, W2, D_H)
  agg2 = _make_agg_kernel(D_H // 2)(srcc, dst, hs2.reshape(NC * N, D_H // 2))
  hs3 = _mid_layer(agg2, dinv, b2, g2, be2, W3, D_OUT, split=False)
  agg3 = _make_agg3_kernel()(src, dst, hs3)
  return _k4(agg3, hs3, dinv, b3)


# speedup vs baseline: 7.6753x; 1.1270x over previous
import functools

import jax
import jax.numpy as jnp
import numpy as np
from jax import lax
from jax.experimental import pallas as pl
from jax.experimental.pallas import tpu as pltpu
from jax.experimental.pallas import tpu_sc as plsc

N = 10000
E = 160000
D_IN = 256
D_H = 256
D_OUT = 128
BN_EPS = 1e-5

NC = 2
NS = 16
STRIPE = 1000
NSTRIPES = N // STRIPE
CHUNK = 80
NB = 4
N_PAD = N + 8

EPT = 10240
EP = NS * EPT
EPW = 5120


def _sc_mesh():
  return plsc.VectorSubcoreMesh(
      core_axis_name="c", subcore_axis_name="s", num_cores=NC, num_subcores=NS
  )


def _agg_pipeline(src_hbm, dst_hbm, hs_hbm, acc, idx_s, idx_d, rows, sems,
                  sem_s, src_base, dst_base, n_chunks):
  def load_idx(j, q):
    pltpu.sync_copy(src_hbm.at[pl.ds(src_base + j * CHUNK, CHUNK)], idx_s.at[q])
    pltpu.sync_copy(dst_hbm.at[pl.ds(dst_base + j * CHUNK, CHUNK)], idx_d.at[q])

  def start_gather(q):
    pltpu.async_copy(hs_hbm.at[idx_s.at[q]], rows.at[q], sems[q])

  def wait_gather(q):
    pltpu.make_async_copy(hs_hbm.at[idx_s.at[q]], rows.at[q], sems[q]).wait()

  load_idx(0, 0)
  load_idx(1, 1)
  load_idx(2, 2)
  start_gather(0)
  start_gather(1)

  def body(jj, carry):
    for b in range(NB):
      j = NB * jj + b
      wait_gather(b)
      desc = pltpu.async_copy(rows.at[b], acc.at[idx_d.at[b]], sem_s, add=True)

      @pl.when(j + 3 < n_chunks)
      def _prefetch_idx():
        load_idx(j + 3, (b + 3) % NB)

      desc.wait()

      @pl.when(j + 2 < n_chunks)
      def _next_gather():
        start_gather((b + 2) % NB)

    return carry

  lax.fori_loop(0, n_chunks // NB, body, 0)


@functools.lru_cache(maxsize=None)
def _make_deg_kernel():
  n_chunks = EPW // CHUNK

  @functools.partial(
      pl.kernel,
      out_type=jax.ShapeDtypeStruct((NC, N, 128), jnp.float32),
      mesh=_sc_mesh(),
      scratch_types=[
          pltpu.VMEM((2, CHUNK), jnp.int32),
          pltpu.VMEM((CHUNK, 128), jnp.float32),
          pltpu.VMEM_SHARED((N_PAD, 128), jnp.float32),
          pltpu.SemaphoreType.DMA,
      ],
  )
  def deg_kernel(dst_hbm, zeros_hbm, ones_hbm, out_hbm, idx_d, ones_v, acc, sem_s):
    c = lax.axis_index("c")
    s = lax.axis_index("s")
    pltpu.sync_copy(ones_hbm, ones_v)

    @pl.when(s < NSTRIPES)
    def _init():
      pltpu.sync_copy(
          zeros_hbm.at[pl.ds(s * STRIPE, STRIPE)],
          acc.at[pl.ds(s * STRIPE, STRIPE)],
      )

    plsc.subcore_barrier()
    base = (c * NS + s) * EPW

    def load_idx(j, q):
      pltpu.sync_copy(dst_hbm.at[pl.ds(base + j * CHUNK, CHUNK)], idx_d.at[q])

    load_idx(0, 0)

    def body(jj, carry):
      for b in range(2):
        j = 2 * jj + b
        desc = pltpu.async_copy(ones_v, acc.at[idx_d.at[b]], sem_s, add=True)

        @pl.when(j + 1 < n_chunks)
        def _prefetch():
          load_idx(j + 1, (b + 1) % 2)

        desc.wait()
      return carry

    lax.fori_loop(0, n_chunks // 2, body, 0)
    plsc.subcore_barrier()

    @pl.when(s < NSTRIPES)
    def _out():
      pltpu.sync_copy(
          acc.at[pl.ds(s * STRIPE, STRIPE)],
          out_hbm.at[c, pl.ds(s * STRIPE, STRIPE)],
      )

  return deg_kernel


def _agg_scratch(half):
  return [
      pltpu.VMEM((NB, CHUNK), jnp.int32),
      pltpu.VMEM((NB, CHUNK), jnp.int32),
      pltpu.VMEM((NB, CHUNK, half), jnp.float32),
      pltpu.VMEM_SHARED((N_PAD, half), jnp.float32),
      pltpu.SemaphoreType.DMA,
      pltpu.SemaphoreType.DMA,
      pltpu.SemaphoreType.DMA,
      pltpu.SemaphoreType.DMA,
      pltpu.SemaphoreType.DMA,
  ]


@functools.lru_cache(maxsize=None)
def _make_agg_kernel(half):
  n_chunks = EPT // CHUNK

  @functools.partial(
      pl.kernel,
      out_type=jax.ShapeDtypeStruct((NC, N, half), jnp.float32),
      mesh=_sc_mesh(),
      scratch_types=_agg_scratch(half),
  )
  def agg_kernel(srcc_hbm, dst_hbm, hs_hbm, out_hbm, idx_s, idx_d, rows, acc,
                 sg0, sg1, sg2, sg3, sem_s):
    c = lax.axis_index("c")
    s = lax.axis_index("s")
    @pl.when(s < NSTRIPES)
    def _init():
      pltpu.sync_copy(
          hs_hbm.at[pl.ds(c * N + s * STRIPE, STRIPE)],
          acc.at[pl.ds(s * STRIPE, STRIPE)],
      )

    plsc.subcore_barrier()
    _agg_pipeline(srcc_hbm, dst_hbm, hs_hbm, acc, idx_s, idx_d, rows,
                  (sg0, sg1, sg2, sg3), sem_s,
                  src_base=c * EP + s * EPT, dst_base=s * EPT,
                  n_chunks=n_chunks)
    plsc.subcore_barrier()

    @pl.when(s < NSTRIPES)
    def _out():
      pltpu.sync_copy(
          acc.at[pl.ds(s * STRIPE, STRIPE)],
          out_hbm.at[c, pl.ds(s * STRIPE, STRIPE)],
      )

  return agg_kernel


@functools.lru_cache(maxsize=None)
def _make_agg3_kernel():
  n_chunks = EPW // CHUNK

  @functools.partial(
      pl.kernel,
      out_type=jax.ShapeDtypeStruct((NC, N, D_OUT), jnp.float32),
      mesh=_sc_mesh(),
      scratch_types=_agg_scratch(D_OUT),
  )
  def agg3_kernel(src_hbm, dst_hbm, hs_hbm, out_hbm, idx_s, idx_d, rows, acc,
                  sg0, sg1, sg2, sg3, sem_s):
    c = lax.axis_index("c")
    s = lax.axis_index("s")

    @pl.when(s < NSTRIPES)
    def _init():
      pltpu.sync_copy(
          hs_hbm.at[pl.ds(s * STRIPE, STRIPE)],
          acc.at[pl.ds(s * STRIPE, STRIPE)],
      )

    plsc.subcore_barrier()
    base = (c * NS + s) * EPW
    _agg_pipeline(src_hbm, dst_hbm, hs_hbm, acc, idx_s, idx_d, rows,
                  (sg0, sg1, sg2, sg3), sem_s,
                  src_base=base, dst_base=base, n_chunks=n_chunks)
    plsc.subcore_barrier()

    @pl.when(s < NSTRIPES)
    def _out():
      pltpu.sync_copy(
          acc.at[pl.ds(s * STRIPE, STRIPE)],
          out_hbm.at[c, pl.ds(s * STRIPE, STRIPE)],
      )

  return agg3_kernel


_BN_ROWS = 1000


def _k1_body(x_ref, w_ref, parts_ref, hs_ref, dinv_ref):
  deg = parts_ref[0, :, 0:1] + parts_ref[1, :, 0:1] + 1.0
  dv = lax.rsqrt(jnp.maximum(deg, 1.0))
  h = lax.dot_general(
      x_ref[...], w_ref[...], (((1,), (0,)), ((), ())),
      preferred_element_type=jnp.float32,
  )
  hs = h * dv
  hs_ref[0] = hs[:, :128]
  hs_ref[1] = hs[:, 128:]
  dinv_ref[...] = dv


def _k1(x, w1, parts):
  g = N // _BN_ROWS
  return pl.pallas_call(
      _k1_body,
      grid=(g,),
      in_specs=[
          pl.BlockSpec((_BN_ROWS, D_IN), lambda i: (i, 0)),
          pl.BlockSpec((D_IN, D_H), lambda i: (0, 0)),
          pl.BlockSpec((NC, _BN_ROWS, 128), lambda i: (0, i, 0)),
      ],
      out_specs=[
          pl.BlockSpec((NC, _BN_ROWS, D_H // 2), lambda i: (0, i, 0)),
          pl.BlockSpec((_BN_ROWS, 1), lambda i: (i, 0)),
      ],
      out_shape=[
          jax.ShapeDtypeStruct((NC, N, D_H // 2), jnp.float32),
          jax.ShapeDtypeStruct((N, 1), jnp.float32),
      ],
  )(x, w1, parts)


def _make_mid_body(d_out, split):
  half = d_out // 2
  bn_c = float(1.0 / np.sqrt(np.float32(1.0 + BN_EPS), dtype=np.float32))

  def body(agg_ref, dinv_ref, b_ref, g_ref, be_ref, w_ref, hs_ref):
    dv = dinv_ref[...]
    a = jnp.concatenate([agg_ref[0], agg_ref[1]], axis=1)
    prev = a * dv + b_ref[...]
    t = prev * (g_ref[...] * bn_c) + be_ref[...]
    h = jnp.maximum(t, 0.0)
    hs = lax.dot_general(
        h, w_ref[...], (((1,), (0,)), ((), ())),
        preferred_element_type=jnp.float32,
    ) * dv
    if split:
      hs_ref[0] = hs[:, :half]
      hs_ref[1] = hs[:, half:]
    else:
      hs_ref[...] = hs

  return body


def _mid_layer(agg, dinv, b, gm, be, w, d_out, split=True):
  g = N // _BN_ROWS
  half = d_out // 2
  if split:
    out_specs = pl.BlockSpec((NC, _BN_ROWS, half), lambda i: (0, i, 0))
    out_shape = jax.ShapeDtypeStruct((NC, N, half), jnp.float32)
  else:
    out_specs = pl.BlockSpec((_BN_ROWS, d_out), lambda i: (i, 0))
    out_shape = jax.ShapeDtypeStruct((N, d_out), jnp.float32)
  return pl.pallas_call(
      _make_mid_body(d_out, split),
      grid=(g,),
      in_specs=[
          pl.BlockSpec((NC, _BN_ROWS, D_H // 2), lambda i: (0, i, 0)),
          pl.BlockSpec((_BN_ROWS, 1), lambda i: (i, 0)),
          pl.BlockSpec((1, D_H), lambda i: (0, 0)),
          pl.BlockSpec((1, D_H), lambda i: (0, 0)),
          pl.BlockSpec((1, D_H), lambda i: (0, 0)),
          pl.BlockSpec((D_H, d_out), lambda i: (0, 0)),
      ],
      out_specs=out_specs,
      out_shape=out_shape,
  )(agg, dinv, b.reshape(1, -1), gm.reshape(1, -1), be.reshape(1, -1), w)


def _k4_body(agg_ref, hs3_ref, dinv_ref, b_ref, out_ref):
  a = agg_ref[0] + agg_ref[1] - hs3_ref[...]
  out_ref[...] = a * dinv_ref[...] + b_ref[...]


def _k4(agg, hs3, dinv, b3):
  g = N // _BN_ROWS
  return pl.pallas_call(
      _k4_body,
      grid=(g,),
      in_specs=[
          pl.BlockSpec((NC, _BN_ROWS, D_OUT), lambda i: (0, i, 0)),
          pl.BlockSpec((_BN_ROWS, D_OUT), lambda i: (i, 0)),
          pl.BlockSpec((_BN_ROWS, 1), lambda i: (i, 0)),
          pl.BlockSpec((1, D_OUT), lambda i: (0, 0)),
      ],
      out_specs=pl.BlockSpec((_BN_ROWS, D_OUT), lambda i: (i, 0)),
      out_shape=jax.ShapeDtypeStruct((N, D_OUT), jnp.float32),
  )(agg, hs3, dinv, b3.reshape(1, -1))


def kernel(x, edge_index, W1, b1, g1, be1, W2, b2, g2, be2, W3, b3):
  src = edge_index[0]
  dst = edge_index[1]
  ept0 = E // NS
  epw0 = E // (NC * NS)
  src_p = jnp.pad(src.reshape(NS, ept0), ((0, 0), (0, EPT - ept0))).reshape(-1)
  dst_p = jnp.pad(dst.reshape(NS, ept0), ((0, 0), (0, EPT - ept0)),
                  constant_values=N).reshape(-1)
  srcc_p = jnp.concatenate([src_p, src_p + N])
  src3_p = jnp.pad(src.reshape(NC * NS, epw0),
                   ((0, 0), (0, EPW - epw0))).reshape(-1)
  dst3_p = jnp.pad(dst.reshape(NC * NS, epw0), ((0, 0), (0, EPW - epw0)),
                   constant_values=N).reshape(-1)
  zeros128 = jnp.zeros((N, 128), jnp.float32)
  ones128 = jnp.ones((CHUNK, 128), jnp.float32)

  parts = _make_deg_kernel()(dst3_p, zeros128, ones128)
  hs1, dinv = _k1(x, W1, parts)
  agg1 = _make_agg_kernel(D_H // 2)(srcc_p, dst_p, hs1.reshape(NC * N, D_H // 2))
  hs2 = _mid_layer(agg1, dinv, b1, g1, be1, W2, D_H)
  agg2 = _make_agg_kernel(D_H // 2)(srcc_p, dst_p, hs2.reshape(NC * N, D_H // 2))
  hs3 = _mid_layer(agg2, dinv, b2, g2, be2, W3, D_OUT, split=False)
  agg3 = _make_agg3_kernel()(src3_p, dst3_p, hs3)
  return _k4(agg3, hs3, dinv, b3)
